# baseline (speedup 1.0000x reference)
import functools

import jax
import jax.numpy as jnp
from jax import lax
from jax.experimental import pallas as pl
from jax.experimental.pallas import tpu as pltpu
from jax.experimental.pallas import tpu_sc as plsc

BATCH = 4096
SLOT_NUM = 26
EMBED_DIM = 32
VOCAB = 1000000
TOTAL_NNZ = 1064960
NUM_ROWS = BATCH * SLOT_NUM

NC = 2
NS = 16
L = 16

ROWS_PER_SC = NUM_ROWS // NC
ROWS_PER_SUB = ROWS_PER_SC // NS
DUMMY_ROW = ROWS_PER_SC
ACC_ROWS = ROWS_PER_SC + 8

G = 128
SG = 8
NGROUPS = TOTAL_NNZ // G
BIG = 1 << 28


def _sc_body(table, vals1d, rows1d, bounds, out,
             vals_s, rows_s, rel_s, gath, bnd_v, acc, sem):
    cid = lax.axis_index("c")
    sid = lax.axis_index("s")

    pltpu.sync_copy(bounds, bnd_v)
    bv = bnd_v[pl.ds(0, L)]
    s = jnp.where(cid == 0, bv[0], bv[1])
    e = jnp.where(cid == 0, bv[1], bv[2])
    base = cid * ROWS_PER_SC

    g0 = (s // G) // SG * SG
    g1 = jnp.minimum(((e + (G - 1)) // G + (SG - 1)) // SG * SG, NGROUPS)
    gq = (g1 - g0 + (NS - 1)) // NS
    gq = (gq + (SG - 1)) // SG * SG
    ga = g0 + sid * gq
    gb = jnp.minimum(ga + gq, g1)
    nst = jnp.maximum((gb - ga + (SG - 1)) // SG, 0)

    def _zero(f, c):
        gath[f >> 1, pl.ds((f & 1) * L, L)] = jnp.zeros((L,), jnp.float32)
        return c
    lax.fori_loop(0, 2 * G, _zero, 0)
    row0 = sid * ROWS_PER_SUB
    for kk in range(ROWS_PER_SUB // G):
        pltpu.sync_copy(gath, acc.at[pl.ds(row0 + kk * G, G)])
    plsc.subcore_barrier()

    def _stage(t, c):
        gs = pl.multiple_of(ga + t * SG, SG)
        pltpu.sync_copy(vals1d.at[pl.ds(gs * G, SG * G)], vals_s)
        pltpu.sync_copy(rows1d.at[pl.ds(gs * G, SG * G)], rows_s)

        def _rel(f, c2):
            j = f >> 3
            i = f & 7
            r = rows_s[pl.ds(f * L, L)]
            off = jnp.where(gs + j < gb, 0, BIG) - base
            rel = r + off
            ok = (rel >= 0) & (rel < ROWS_PER_SC)
            rel_s[j, pl.ds(i * L, L)] = jnp.where(ok, rel, DUMMY_ROW)
            return c2
        lax.fori_loop(0, SG * (G // L), _rel, 0)

        for j in range(SG):
            pltpu.async_copy(table.at[vals_s.at[pl.ds(j * G, G)]], gath,
                             sem).wait()
            pltpu.sync_copy(gath, acc.at[rel_s.at[j]], add=True)
        return c
    lax.fori_loop(0, nst, _stage, 0)
    plsc.subcore_barrier()

    for kk in range(ROWS_PER_SUB // G):
        pltpu.sync_copy(acc.at[pl.ds(row0 + kk * G, G)], gath)
        pltpu.sync_copy(gath, out.at[pl.ds(base + row0 + kk * G, G)])


@jax.jit
def _sc_lookup(table, vals2d, rows2d, bounds):
    mesh = plsc.VectorSubcoreMesh(core_axis_name="c", subcore_axis_name="s")
    return pl.kernel(
        _sc_body,
        out_type=jax.ShapeDtypeStruct((NUM_ROWS, EMBED_DIM), jnp.float32),
        mesh=mesh,
        compiler_params=pltpu.CompilerParams(use_tc_tiling_on_sc=False),
        scratch_types=[
            pltpu.VMEM((SG * G,), jnp.int32),
            pltpu.VMEM((SG * G,), jnp.int32),
            pltpu.VMEM((SG, G), jnp.int32),
            pltpu.VMEM((G, EMBED_DIM), jnp.float32),
            pltpu.VMEM((L,), jnp.int32),
            pltpu.VMEM_SHARED((ACC_ROWS, EMBED_DIM), jnp.float32),
            pltpu.SemaphoreType.DMA,
        ],
    )(table, vals2d, rows2d, bounds)


def kernel(values, row_indices, table):
    split = jnp.searchsorted(row_indices, ROWS_PER_SC).astype(jnp.int32)
    bounds = jnp.zeros((L,), jnp.int32).at[1].set(split).at[2].set(TOTAL_NNZ)
    out = _sc_lookup(table, values, row_indices, bounds)
    return out.reshape(BATCH, SLOT_NUM, EMBED_DIM)


# speedup vs baseline: 1.3357x; 1.3357x over previous
import functools

import jax
import jax.numpy as jnp
from jax import lax
from jax.experimental import pallas as pl
from jax.experimental.pallas import tpu as pltpu
from jax.experimental.pallas import tpu_sc as plsc

BATCH = 4096
SLOT_NUM = 26
EMBED_DIM = 32
VOCAB = 1000000
TOTAL_NNZ = 1064960
NUM_ROWS = BATCH * SLOT_NUM

NC = 2
NS = 16
L = 16

ROWS_PER_SC = NUM_ROWS // NC
ROWS_PER_SUB = ROWS_PER_SC // NS
DUMMY_ROW = ROWS_PER_SC
ACC_ROWS = ROWS_PER_SC + 8

G = 128
SG = 8
NBUF = 4
LA = 3
NGROUPS = TOTAL_NNZ // G
BIG = 1 << 28


def _sc_body(table, vals1d, rows1d, bounds, out,
             vals_s, rows_s, rel_s, gath, bnd_v, acc,
             sem_g, sem_a, sem_l):
    cid = lax.axis_index("c")
    sid = lax.axis_index("s")

    pltpu.sync_copy(bounds, bnd_v)
    bv = bnd_v[pl.ds(0, L)]
    s = jnp.where(cid == 0, bv[0], bv[1])
    e = jnp.where(cid == 0, bv[1], bv[2])
    base = cid * ROWS_PER_SC

    g0 = (s // G) // SG * SG
    g1 = jnp.minimum(((e + (G - 1)) // G + (SG - 1)) // SG * SG, NGROUPS)
    gq = (g1 - g0 + (NS - 1)) // NS
    gq = (gq + (SG - 1)) // SG * SG
    ga = g0 + sid * gq
    gb = jnp.minimum(ga + gq, g1)
    nst = jnp.maximum((gb - ga + (SG - 1)) // SG, 0)

    def _zero(f, c):
        gath[f >> 1, pl.ds((f & 1) * L, L)] = jnp.zeros((L,), jnp.float32)
        return c
    lax.fori_loop(0, 2 * G, _zero, 0)
    row0 = sid * ROWS_PER_SUB
    zsrc = gath.at[pl.ds(0, G)]
    for kk in range(ROWS_PER_SUB // G):
        pltpu.sync_copy(zsrc, acc.at[pl.ds(row0 + kk * G, G)])
    plsc.subcore_barrier()

    ngr = nst * SG

    def _rel_compute(t, par):
        gs = ga + t * SG
        pbase = par * SG * G

        def _rel(f, c2):
            j = f >> 3
            i = f & 7
            r = rows_s[pl.ds(pbase + f * L, L)]
            off = jnp.where(gs + j < gb, 0, BIG) - base
            rel = r + off
            ok = (rel >= 0) & (rel < ROWS_PER_SC)
            rel_s[par * SG + j, pl.ds(i * L, L)] = jnp.where(ok, rel,
                                                             DUMMY_ROW)
            return c2
        lax.fori_loop(0, SG * (G // L), _rel, 0)

    def _fire_gather(gg):
        tg = gg // SG
        voff = (tg % 2) * SG * G + (gg % SG) * G
        pltpu.async_copy(table.at[vals_s.at[pl.ds(voff, G)]],
                         gath.at[pl.ds((gg % NBUF) * G, G)], sem_g)

    def _drain1(sem):
        pltpu.make_async_copy(table.at[pl.ds(0, G)],
                              gath.at[pl.ds(0, G)], sem).wait()

    @pl.when(nst > 0)
    def _prologue():
        gs0 = pl.multiple_of(ga, SG) * G
        pltpu.sync_copy(vals1d.at[pl.ds(gs0, SG * G)],
                        vals_s.at[pl.ds(0, SG * G)])
        pltpu.sync_copy(rows1d.at[pl.ds(gs0, SG * G)],
                        rows_s.at[pl.ds(0, SG * G)])
        _rel_compute(0, 0)
        for k in range(LA):
            @pl.when(k < ngr)
            def _f():
                _fire_gather(jnp.int32(k))

    def _group(g, c):
        t = g // SG
        j = g % SG
        par = t % 2

        @pl.when((j == 0) & (t + 1 < nst))
        def _fire_loads():
            gsn = pl.multiple_of(ga + (t + 1) * SG, SG)
            nxt = 1 - par
            pltpu.async_copy(vals1d.at[pl.ds(gsn * G, SG * G)],
                             vals_s.at[pl.ds(nxt * SG * G, SG * G)], sem_l)
            pltpu.async_copy(rows1d.at[pl.ds(gsn * G, SG * G)],
                             rows_s.at[pl.ds(nxt * SG * G, SG * G)], sem_l)

        @pl.when((j == 0) & (t > 0))
        def _rel_this_stage():
            _rel_compute(t, par)

        @pl.when((j == 4) & (t + 1 < nst))
        def _wait_loads():
            pltpu.make_async_copy(vals1d.at[pl.ds(0, SG * G)],
                                  vals_s.at[pl.ds(0, SG * G)], sem_l).wait()
            pltpu.make_async_copy(rows1d.at[pl.ds(0, SG * G)],
                                  rows_s.at[pl.ds(0, SG * G)], sem_l).wait()

        _drain1(sem_g)
        pltpu.async_copy(gath.at[pl.ds((g % NBUF) * G, G)],
                         acc.at[rel_s.at[par * SG + j]], sem_a, add=True)

        @pl.when(g >= 1)
        def _drain_scatter():
            _drain1(sem_a)

        @pl.when(g + LA < ngr)
        def _fire_next():
            _fire_gather(g + LA)
        return c
    lax.fori_loop(0, ngr, _group, 0)

    @pl.when(nst > 0)
    def _epilogue():
        _drain1(sem_a)
    plsc.subcore_barrier()

    for kk in range(ROWS_PER_SUB // G):
        pltpu.sync_copy(acc.at[pl.ds(row0 + kk * G, G)], zsrc)
        pltpu.sync_copy(zsrc, out.at[pl.ds(base + row0 + kk * G, G)])


@jax.jit
def _sc_lookup(table, vals2d, rows2d, bounds):
    mesh = plsc.VectorSubcoreMesh(core_axis_name="c", subcore_axis_name="s")
    return pl.kernel(
        _sc_body,
        out_type=jax.ShapeDtypeStruct((NUM_ROWS, EMBED_DIM), jnp.float32),
        mesh=mesh,
        compiler_params=pltpu.CompilerParams(use_tc_tiling_on_sc=False),
        scratch_types=[
            pltpu.VMEM((2 * SG * G,), jnp.int32),
            pltpu.VMEM((2 * SG * G,), jnp.int32),
            pltpu.VMEM((2 * SG, G), jnp.int32),
            pltpu.VMEM((NBUF * G, EMBED_DIM), jnp.float32),
            pltpu.VMEM((L,), jnp.int32),
            pltpu.VMEM_SHARED((ACC_ROWS, EMBED_DIM), jnp.float32),
            pltpu.SemaphoreType.DMA,
            pltpu.SemaphoreType.DMA,
            pltpu.SemaphoreType.DMA,
        ],
    )(table, vals2d, rows2d, bounds)


def kernel(values, row_indices, table):
    split = jnp.searchsorted(row_indices, ROWS_PER_SC).astype(jnp.int32)
    bounds = jnp.zeros((L,), jnp.int32).at[1].set(split).at[2].set(TOTAL_NNZ)
    out = _sc_lookup(table, values, row_indices, bounds)
    return out.reshape(BATCH, SLOT_NUM, EMBED_DIM)
